# Initial kernel scaffold; baseline (speedup 1.0000x reference)
import jax
import jax.numpy as jnp
from jax.experimental import pallas as pl


def kernel(x, edge_index, W, b):
    raise NotImplementedError("write your pallas kernel here")


# speedup vs baseline: 15.7794x; 15.7794x over previous
import functools

import jax
import jax.numpy as jnp
from jax import lax
from jax.experimental import pallas as pl
from jax.experimental.pallas import tpu as pltpu
from jax.experimental.pallas import tpu_sc as plsc

N = 10000
D = 128
C = 40
E = 320000

NP = 10240
CP = 48
NC = 2
NS = 16
NW = NC * NS
CH = 128
CHUNKS = 79
EPT = CHUNKS * CH
EP = NW * EPT
RPS = NP // NS

_mesh = plsc.VectorSubcoreMesh(core_axis_name="c", subcore_axis_name="s")


@functools.partial(
    pl.kernel,
    out_type=jax.ShapeDtypeStruct((NW, NP), jnp.float32),
    mesh=_mesh,
    scratch_types=[
        pltpu.VMEM((CH,), jnp.int32),
        pltpu.VMEM((NP,), jnp.float32),
    ],
    compiler_params=pltpu.CompilerParams(needs_layout_passes=False),
)
def _deg(dst_hbm, out_hbm, idx_v, acc_v):
    c = lax.axis_index("c")
    s = lax.axis_index("s")
    wid = s * NC + c

    def zero_body(i, carry):
        acc_v[pl.ds(i * 16, 16)] = jnp.zeros((16,), jnp.float32)
        return carry

    lax.fori_loop(0, NP // 16, zero_body, 0)

    ones = jnp.ones((16,), jnp.float32)
    base = wid * EPT

    def chunk_body(j, carry):
        pltpu.sync_copy(dst_hbm.at[pl.ds(base + j * CH, CH)], idx_v)

        def vec_body(k, inner):
            dvec = idx_v[pl.ds(k * 16, 16)]
            plsc.addupdate_scatter(acc_v, [dvec], ones)
            return inner

        lax.fori_loop(0, CH // 16, vec_body, 0)
        return carry

    lax.fori_loop(0, CHUNKS, chunk_body, 0)
    pltpu.sync_copy(acc_v, out_hbm.at[wid])


@functools.partial(
    pl.kernel,
    out_type=jax.ShapeDtypeStruct((NC, NP, CP), jnp.float32),
    mesh=_mesh,
    scratch_types=[
        pltpu.VMEM((CH,), jnp.int32),
        pltpu.VMEM((CH,), jnp.int32),
        pltpu.VMEM((CH, CP), jnp.float32),
        pltpu.VMEM((CH, CP), jnp.float32),
        pltpu.VMEM_SHARED((NP, CP), jnp.float32),
        pltpu.SemaphoreType.DMA,
    ],
    compiler_params=pltpu.CompilerParams(use_tc_tiling_on_sc=False),
)
def _hop(src_hbm, dst_hbm, g_hbm, out_hbm, si_v, di_v, rows_v, zero_v,
         acc_sh, sem):
    c = lax.axis_index("c")
    s = lax.axis_index("s")
    wid = s * NC + c

    def zrow(i, carry):
        def zcol(j, inner):
            zero_v[i, pl.ds(j * 16, 16)] = jnp.zeros((16,), jnp.float32)
            return inner

        lax.fori_loop(0, CP // 16, zcol, 0)
        return carry

    lax.fori_loop(0, CH, zrow, 0)

    def zacc(i, carry):
        pltpu.sync_copy(zero_v, acc_sh.at[pl.ds(s * RPS + i * CH, CH)])
        return carry

    lax.fori_loop(0, RPS // CH, zacc, 0)
    plsc.subcore_barrier()

    base = wid * EPT

    def chunk_body(j, carry):
        off = base + j * CH
        pltpu.sync_copy(src_hbm.at[pl.ds(off, CH)], si_v)
        pltpu.sync_copy(dst_hbm.at[pl.ds(off, CH)], di_v)
        pltpu.async_copy(g_hbm.at[si_v], rows_v, sem).wait()
        pltpu.sync_copy(rows_v, acc_sh.at[di_v], add=True)
        return carry

    lax.fori_loop(0, CHUNKS, chunk_body, 0)
    plsc.subcore_barrier()

    pltpu.sync_copy(acc_sh.at[pl.ds(s * RPS, RPS)],
                    out_hbm.at[c, pl.ds(s * RPS, RPS)])


_BR = 256


def _k1_body(x_ref, wt_ref, dp_ref, g0_ref):
    z = jnp.dot(x_ref[...], wt_ref[...], preferred_element_type=jnp.float32)
    deg = jnp.sum(dp_ref[...], axis=0) + 1.0
    g0_ref[...] = z * lax.rsqrt(deg)[:, None]


def _k1(xp, wt, dp):
    return pl.pallas_call(
        _k1_body,
        grid=(NP // _BR,),
        in_specs=[
            pl.BlockSpec((_BR, D), lambda i: (i, 0)),
            pl.BlockSpec((D, CP), lambda i: (0, 0)),
            pl.BlockSpec((NW, _BR), lambda i: (0, i)),
        ],
        out_specs=pl.BlockSpec((_BR, CP), lambda i: (i, 0)),
        out_shape=jax.ShapeDtypeStruct((NP, CP), jnp.float32),
    )(xp, wt, dp)


def _k2_body(sp_ref, g_ref, dp_ref, o_ref):
    deg = jnp.sum(dp_ref[...], axis=0) + 1.0
    t = sp_ref[0] + sp_ref[1] + g_ref[...]
    o_ref[...] = t / deg[:, None]


def _k2(sp, g, dp):
    return pl.pallas_call(
        _k2_body,
        grid=(NP // _BR,),
        in_specs=[
            pl.BlockSpec((NC, _BR, CP), lambda i: (0, i, 0)),
            pl.BlockSpec((_BR, CP), lambda i: (i, 0)),
            pl.BlockSpec((NW, _BR), lambda i: (0, i)),
        ],
        out_specs=pl.BlockSpec((_BR, CP), lambda i: (i, 0)),
        out_shape=jax.ShapeDtypeStruct((NP, CP), jnp.float32),
    )(sp, g, dp)


def _k3_body(sp_ref, g_ref, dp_ref, b_ref, o_ref):
    deg = jnp.sum(dp_ref[...], axis=0) + 1.0
    dis = lax.rsqrt(deg)
    v = (sp_ref[0] + sp_ref[1] + g_ref[...]) * dis[:, None] + b_ref[...]
    col = lax.broadcasted_iota(jnp.int32, v.shape, 1)
    mask = col < C
    vm = jnp.where(mask, v, -jnp.inf)
    m = jnp.max(vm, axis=1, keepdims=True)
    ex = jnp.where(mask, jnp.exp(v - m), 0.0)
    lse = jnp.log(jnp.sum(ex, axis=1, keepdims=True)) + m
    o_ref[...] = v - lse


def _k3(sp, g, dp, b2):
    return pl.pallas_call(
        _k3_body,
        grid=(NP // _BR,),
        in_specs=[
            pl.BlockSpec((NC, _BR, CP), lambda i: (0, i, 0)),
            pl.BlockSpec((_BR, CP), lambda i: (i, 0)),
            pl.BlockSpec((NW, _BR), lambda i: (0, i)),
            pl.BlockSpec((1, CP), lambda i: (0, 0)),
        ],
        out_specs=pl.BlockSpec((_BR, CP), lambda i: (i, 0)),
        out_shape=jax.ShapeDtypeStruct((NP, CP), jnp.float32),
    )(sp, g, dp, b2)


def kernel(x, edge_index, W, b):
    src = edge_index[0]
    dst = edge_index[1]
    pad = jnp.full((EP - E,), NP - 1, dtype=jnp.int32)
    srcp = jnp.concatenate([src, pad])
    dstp = jnp.concatenate([dst, pad])
    xp = jnp.pad(x, ((0, NP - N), (0, 0)))
    wt = jnp.pad(W.T, ((0, 0), (0, CP - C)))
    b2 = jnp.pad(b, (0, CP - C)).reshape(1, CP)

    dp = _deg(dstp)
    g0 = _k1(xp, wt, dp)
    s1 = _hop(srcp, dstp, g0)
    g1 = _k2(s1, g0, dp)
    s2 = _hop(srcp, dstp, g1)
    out = _k3(s2, g1, dp, b2)
    return out[:N, :C]


# speedup vs baseline: 19.5090x; 1.2364x over previous
import functools

import jax
import jax.numpy as jnp
from jax import lax
from jax.experimental import pallas as pl
from jax.experimental.pallas import tpu as pltpu
from jax.experimental.pallas import tpu_sc as plsc

N = 10000
D = 128
C = 40
E = 320000

NP = 10240
CP = 48
NC = 2
NS = 16
NW = NC * NS
CH = 128
CHUNKS = 80
EPT = CHUNKS * CH
EP = NW * EPT
RPS = NP // NS
NBUF = 4
GRP = CHUNKS // NBUF

_mesh = plsc.VectorSubcoreMesh(core_axis_name="c", subcore_axis_name="s")


@functools.partial(
    pl.kernel,
    out_type=jax.ShapeDtypeStruct((NW, NP), jnp.float32),
    mesh=_mesh,
    scratch_types=[
        pltpu.VMEM((CHUNKS, CH), jnp.int32),
        pltpu.VMEM((NP,), jnp.float32),
    ],
    compiler_params=pltpu.CompilerParams(needs_layout_passes=False),
)
def _deg(dst_hbm, out_hbm, idx_v, acc_v):
    c = lax.axis_index("c")
    s = lax.axis_index("s")
    wid = s * NC + c

    def zero_body(i, carry):
        acc_v[pl.ds(i * 16, 16)] = jnp.zeros((16,), jnp.float32)
        return carry

    lax.fori_loop(0, NP // 16, zero_body, 0)
    pltpu.sync_copy(dst_hbm.at[wid], idx_v)

    ones = jnp.ones((16,), jnp.float32)

    def chunk_body(j, carry):
        def vec_body(k, inner):
            dvec = idx_v[j, pl.ds(k * 16, 16)]
            plsc.addupdate_scatter(acc_v, [dvec], ones)
            return inner

        lax.fori_loop(0, CH // 16, vec_body, 0)
        return carry

    lax.fori_loop(0, CHUNKS, chunk_body, 0)
    pltpu.sync_copy(acc_v, out_hbm.at[wid])


@functools.partial(
    pl.kernel,
    out_type=jax.ShapeDtypeStruct((NC, NP, CP), jnp.float32),
    mesh=_mesh,
    scratch_types=[
        pltpu.VMEM((EPT,), jnp.int32),
        pltpu.VMEM((CHUNKS, CH), jnp.int32),
        [pltpu.VMEM((CH, CP), jnp.float32)] * NBUF,
        pltpu.VMEM((CH, CP), jnp.float32),
        pltpu.VMEM_SHARED((NP, CP), jnp.float32),
        [pltpu.SemaphoreType.DMA] * NBUF,
        [pltpu.SemaphoreType.DMA] * NBUF,
    ],
    compiler_params=pltpu.CompilerParams(use_tc_tiling_on_sc=False),
)
def _hop(src_hbm, dst_hbm, g_hbm, out_hbm, sidx_v, didx_v, rows, zero_v,
         acc_sh, sg, ss):
    c = lax.axis_index("c")
    s = lax.axis_index("s")
    wid = s * NC + c

    pltpu.sync_copy(src_hbm.at[pl.ds(wid * EPT, EPT)], sidx_v)
    pltpu.sync_copy(dst_hbm.at[wid], didx_v)

    def gstart(b, j):
        pltpu.async_copy(g_hbm.at[sidx_v.at[pl.ds(j * CH, CH)]], rows[b],
                         sg[b])

    def gwait(b):
        pltpu.make_async_copy(g_hbm.at[sidx_v.at[pl.ds(0, CH)]], rows[b],
                              sg[b]).wait()

    def sstart(b, j):
        pltpu.async_copy(rows[b], acc_sh.at[didx_v.at[j]], ss[b], add=True)

    def swait(b):
        pltpu.make_async_copy(rows[b], acc_sh.at[didx_v.at[0]],
                              ss[b]).wait()

    gstart(0, 0)
    gstart(1, 1)

    def zrow(i, carry):
        def zcol(jj, inner):
            zero_v[i, pl.ds(jj * 16, 16)] = jnp.zeros((16,), jnp.float32)
            return inner

        lax.fori_loop(0, CP // 16, zcol, 0)
        return carry

    lax.fori_loop(0, CH, zrow, 0)

    def zacc(i, carry):
        pltpu.sync_copy(zero_v, acc_sh.at[pl.ds(s * RPS + i * CH, CH)])
        return carry

    lax.fori_loop(0, RPS // CH, zacc, 0)
    plsc.subcore_barrier()

    def group_body(j4, carry):
        for b in range(NBUF):
            j = j4 * NBUF + b
            gwait(b)
            sstart(b, j)
            b2 = (b + 2) % NBUF
            if b < 2:
                @pl.when(j4 > 0)
                def _():
                    swait(b2)

                gstart(b2, j + 2)
            else:
                swait(b2)

                @pl.when(j4 < GRP - 1)
                def _():
                    gstart(b2, j + 2)
        return carry

    lax.fori_loop(0, GRP, group_body, 0)
    swait(2)
    swait(3)
    plsc.subcore_barrier()

    pltpu.sync_copy(acc_sh.at[pl.ds(s * RPS, RPS)],
                    out_hbm.at[c, pl.ds(s * RPS, RPS)])


_BR = 256


def _k1_body(x_ref, wt_ref, dp_ref, g0_ref):
    z = jnp.dot(x_ref[...], wt_ref[...], preferred_element_type=jnp.float32)
    deg = jnp.sum(dp_ref[...], axis=0) + 1.0
    g0_ref[...] = z * lax.rsqrt(deg)[:, None]


def _k1(xp, wt, dp):
    return pl.pallas_call(
        _k1_body,
        grid=(NP // _BR,),
        in_specs=[
            pl.BlockSpec((_BR, D), lambda i: (i, 0)),
            pl.BlockSpec((D, CP), lambda i: (0, 0)),
            pl.BlockSpec((NW, _BR), lambda i: (0, i)),
        ],
        out_specs=pl.BlockSpec((_BR, CP), lambda i: (i, 0)),
        out_shape=jax.ShapeDtypeStruct((NP, CP), jnp.float32),
    )(xp, wt, dp)


def _k2_body(sp_ref, g_ref, dp_ref, o_ref):
    deg = jnp.sum(dp_ref[...], axis=0) + 1.0
    t = sp_ref[0] + sp_ref[1] + g_ref[...]
    o_ref[...] = t / deg[:, None]


def _k2(sp, g, dp):
    return pl.pallas_call(
        _k2_body,
        grid=(NP // _BR,),
        in_specs=[
            pl.BlockSpec((NC, _BR, CP), lambda i: (0, i, 0)),
            pl.BlockSpec((_BR, CP), lambda i: (i, 0)),
            pl.BlockSpec((NW, _BR), lambda i: (0, i)),
        ],
        out_specs=pl.BlockSpec((_BR, CP), lambda i: (i, 0)),
        out_shape=jax.ShapeDtypeStruct((NP, CP), jnp.float32),
    )(sp, g, dp)


def _k3_body(sp_ref, g_ref, dp_ref, b_ref, o_ref):
    deg = jnp.sum(dp_ref[...], axis=0) + 1.0
    dis = lax.rsqrt(deg)
    v = (sp_ref[0] + sp_ref[1] + g_ref[...]) * dis[:, None] + b_ref[...]
    col = lax.broadcasted_iota(jnp.int32, v.shape, 1)
    mask = col < C
    vm = jnp.where(mask, v, -jnp.inf)
    m = jnp.max(vm, axis=1, keepdims=True)
    ex = jnp.where(mask, jnp.exp(v - m), 0.0)
    lse = jnp.log(jnp.sum(ex, axis=1, keepdims=True)) + m
    o_ref[...] = v - lse


def _k3(sp, g, dp, b2):
    return pl.pallas_call(
        _k3_body,
        grid=(NP // _BR,),
        in_specs=[
            pl.BlockSpec((NC, _BR, CP), lambda i: (0, i, 0)),
            pl.BlockSpec((_BR, CP), lambda i: (i, 0)),
            pl.BlockSpec((NW, _BR), lambda i: (0, i)),
            pl.BlockSpec((1, CP), lambda i: (0, 0)),
        ],
        out_specs=pl.BlockSpec((_BR, CP), lambda i: (i, 0)),
        out_shape=jax.ShapeDtypeStruct((NP, CP), jnp.float32),
    )(sp, g, dp, b2)


def kernel(x, edge_index, W, b):
    src = edge_index[0]
    dst = edge_index[1]
    pad = jnp.full((EP - E,), NP - 1, dtype=jnp.int32)
    srcp = jnp.concatenate([src, pad])
    dstp = jnp.concatenate([dst, pad]).reshape(NW, CHUNKS, CH)
    xp = jnp.pad(x, ((0, NP - N), (0, 0)))
    wt = jnp.pad(W.T, ((0, 0), (0, CP - C)))
    b2 = jnp.pad(b, (0, CP - C)).reshape(1, CP)

    dp = _deg(dstp)
    g0 = _k1(xp, wt, dp)
    s1 = _hop(srcp, dstp, g0)
    g1 = _k2(s1, g0, dp)
    s2 = _hop(srcp, dstp, g1)
    out = _k3(s2, g1, dp, b2)
    return out[:N, :C]


# speedup vs baseline: 19.5322x; 1.0012x over previous
import functools

import jax
import jax.numpy as jnp
from jax import lax
from jax.experimental import pallas as pl
from jax.experimental.pallas import tpu as pltpu
from jax.experimental.pallas import tpu_sc as plsc

N = 10000
D = 128
C = 40
E = 320000

NP = 10240
CP = 48
NC = 2
NS = 16
NW = NC * NS
CH = 128
CHUNKS = 80
EPT = CHUNKS * CH
EP = NW * EPT
RPS = NP // NS
NBUF = 8
LA = 4
GRP = CHUNKS // NBUF

_mesh = plsc.VectorSubcoreMesh(core_axis_name="c", subcore_axis_name="s")


@functools.partial(
    pl.kernel,
    out_type=jax.ShapeDtypeStruct((NW, NP), jnp.float32),
    mesh=_mesh,
    scratch_types=[
        pltpu.VMEM((CHUNKS, CH), jnp.int32),
        pltpu.VMEM((NP,), jnp.float32),
    ],
    compiler_params=pltpu.CompilerParams(needs_layout_passes=False),
)
def _deg(dst_hbm, out_hbm, idx_v, acc_v):
    c = lax.axis_index("c")
    s = lax.axis_index("s")
    wid = s * NC + c

    def zero_body(i, carry):
        acc_v[pl.ds(i * 16, 16)] = jnp.zeros((16,), jnp.float32)
        return carry

    lax.fori_loop(0, NP // 16, zero_body, 0)
    pltpu.sync_copy(dst_hbm.at[wid], idx_v)

    ones = jnp.ones((16,), jnp.float32)

    def chunk_body(j, carry):
        def vec_body(k, inner):
            dvec = idx_v[j, pl.ds(k * 16, 16)]
            plsc.addupdate_scatter(acc_v, [dvec], ones)
            return inner

        lax.fori_loop(0, CH // 16, vec_body, 0)
        return carry

    lax.fori_loop(0, CHUNKS, chunk_body, 0)
    pltpu.sync_copy(acc_v, out_hbm.at[wid])


@functools.partial(
    pl.kernel,
    out_type=jax.ShapeDtypeStruct((NC, NP, CP), jnp.float32),
    mesh=_mesh,
    scratch_types=[
        pltpu.VMEM((EPT,), jnp.int32),
        pltpu.VMEM((CHUNKS, CH), jnp.int32),
        [pltpu.VMEM((CH, CP), jnp.float32)] * NBUF,
        pltpu.VMEM((CH, CP), jnp.float32),
        pltpu.VMEM_SHARED((NP, CP), jnp.float32),
        [pltpu.SemaphoreType.DMA] * NBUF,
        [pltpu.SemaphoreType.DMA] * NBUF,
    ],
    compiler_params=pltpu.CompilerParams(use_tc_tiling_on_sc=False),
)
def _hop(src_hbm, dst_hbm, g_hbm, out_hbm, sidx_v, didx_v, rows, zero_v,
         acc_sh, sg, ss):
    c = lax.axis_index("c")
    s = lax.axis_index("s")
    wid = s * NC + c

    pltpu.sync_copy(src_hbm.at[pl.ds(wid * EPT, EPT)], sidx_v)
    pltpu.sync_copy(dst_hbm.at[wid], didx_v)

    def gstart(b, j):
        pltpu.async_copy(g_hbm.at[sidx_v.at[pl.ds(j * CH, CH)]], rows[b],
                         sg[b])

    def gwait(b):
        pltpu.make_async_copy(g_hbm.at[sidx_v.at[pl.ds(0, CH)]], rows[b],
                              sg[b]).wait()

    def sstart(b, j):
        pltpu.async_copy(rows[b], acc_sh.at[didx_v.at[j]], ss[b], add=True)

    def swait(b):
        pltpu.make_async_copy(rows[b], acc_sh.at[didx_v.at[0]],
                              ss[b]).wait()

    for b0 in range(LA):
        gstart(b0, b0)

    def zrow(i, carry):
        def zcol(jj, inner):
            zero_v[i, pl.ds(jj * 16, 16)] = jnp.zeros((16,), jnp.float32)
            return inner

        lax.fori_loop(0, CP // 16, zcol, 0)
        return carry

    lax.fori_loop(0, CH, zrow, 0)

    def zacc(i, carry):
        pltpu.sync_copy(zero_v, acc_sh.at[pl.ds(s * RPS + i * CH, CH)])
        return carry

    lax.fori_loop(0, RPS // CH, zacc, 0)
    plsc.subcore_barrier()

    def group_body(j4, carry):
        for b in range(NBUF):
            j = j4 * NBUF + b
            gwait(b)
            sstart(b, j)
            b2 = (b + LA) % NBUF
            if b < LA:
                @pl.when(j4 > 0)
                def _():
                    swait(b2)

                gstart(b2, j + LA)
            else:
                swait(b2)

                @pl.when(j4 < GRP - 1)
                def _():
                    gstart(b2, j + LA)
        return carry

    lax.fori_loop(0, GRP, group_body, 0)
    for b0 in range(LA, NBUF):
        swait(b0)
    plsc.subcore_barrier()

    pltpu.sync_copy(acc_sh.at[pl.ds(s * RPS, RPS)],
                    out_hbm.at[c, pl.ds(s * RPS, RPS)])


_BR = 256


def _k1_body(x_ref, wt_ref, dp_ref, g0_ref):
    z = jnp.dot(x_ref[...], wt_ref[...], preferred_element_type=jnp.float32)
    deg = jnp.sum(dp_ref[...], axis=0) + 1.0
    g0_ref[...] = z * lax.rsqrt(deg)[:, None]


def _k1(xp, wt, dp):
    return pl.pallas_call(
        _k1_body,
        grid=(NP // _BR,),
        in_specs=[
            pl.BlockSpec((_BR, D), lambda i: (i, 0)),
            pl.BlockSpec((D, CP), lambda i: (0, 0)),
            pl.BlockSpec((NW, _BR), lambda i: (0, i)),
        ],
        out_specs=pl.BlockSpec((_BR, CP), lambda i: (i, 0)),
        out_shape=jax.ShapeDtypeStruct((NP, CP), jnp.float32),
    )(xp, wt, dp)


def _k2_body(sp_ref, g_ref, dp_ref, o_ref):
    deg = jnp.sum(dp_ref[...], axis=0) + 1.0
    t = sp_ref[0] + sp_ref[1] + g_ref[...]
    o_ref[...] = t / deg[:, None]


def _k2(sp, g, dp):
    return pl.pallas_call(
        _k2_body,
        grid=(NP // _BR,),
        in_specs=[
            pl.BlockSpec((NC, _BR, CP), lambda i: (0, i, 0)),
            pl.BlockSpec((_BR, CP), lambda i: (i, 0)),
            pl.BlockSpec((NW, _BR), lambda i: (0, i)),
        ],
        out_specs=pl.BlockSpec((_BR, CP), lambda i: (i, 0)),
        out_shape=jax.ShapeDtypeStruct((NP, CP), jnp.float32),
    )(sp, g, dp)


def _k3_body(sp_ref, g_ref, dp_ref, b_ref, o_ref):
    deg = jnp.sum(dp_ref[...], axis=0) + 1.0
    dis = lax.rsqrt(deg)
    v = (sp_ref[0] + sp_ref[1] + g_ref[...]) * dis[:, None] + b_ref[...]
    col = lax.broadcasted_iota(jnp.int32, v.shape, 1)
    mask = col < C
    vm = jnp.where(mask, v, -jnp.inf)
    m = jnp.max(vm, axis=1, keepdims=True)
    ex = jnp.where(mask, jnp.exp(v - m), 0.0)
    lse = jnp.log(jnp.sum(ex, axis=1, keepdims=True)) + m
    o_ref[...] = v - lse


def _k3(sp, g, dp, b2):
    return pl.pallas_call(
        _k3_body,
        grid=(NP // _BR,),
        in_specs=[
            pl.BlockSpec((NC, _BR, CP), lambda i: (0, i, 0)),
            pl.BlockSpec((_BR, CP), lambda i: (i, 0)),
            pl.BlockSpec((NW, _BR), lambda i: (0, i)),
            pl.BlockSpec((1, CP), lambda i: (0, 0)),
        ],
        out_specs=pl.BlockSpec((_BR, CP), lambda i: (i, 0)),
        out_shape=jax.ShapeDtypeStruct((NP, CP), jnp.float32),
    )(sp, g, dp, b2)


def kernel(x, edge_index, W, b):
    src = edge_index[0]
    dst = edge_index[1]
    pad = jnp.full((EP - E,), NP - 1, dtype=jnp.int32)
    srcp = jnp.concatenate([src, pad])
    dstp = jnp.concatenate([dst, pad]).reshape(NW, CHUNKS, CH)
    xp = jnp.pad(x, ((0, NP - N), (0, 0)))
    wt = jnp.pad(W.T, ((0, 0), (0, CP - C)))
    b2 = jnp.pad(b, (0, CP - C)).reshape(1, CP)

    dp = _deg(dstp)
    g0 = _k1(xp, wt, dp)
    s1 = _hop(srcp, dstp, g0)
    g1 = _k2(s1, g0, dp)
    s2 = _hop(srcp, dstp, g1)
    out = _k3(s2, g1, dp, b2)
    return out[:N, :C]


# speedup vs baseline: 20.3344x; 1.0411x over previous
import functools

import jax
import jax.numpy as jnp
from jax import lax
from jax.experimental import pallas as pl
from jax.experimental.pallas import tpu as pltpu
from jax.experimental.pallas import tpu_sc as plsc

N = 10000
D = 128
C = 40
E = 320000

NP = 10240
CP = 48
NC = 2
NS = 16
NW = NC * NS
CH = 128
CHUNKS = 80
EPT = CHUNKS * CH
EP = NW * EPT
TCH = EP // CH
RPS = NP // NS
NBUF = 8
LA = 4

SLOW_CORE = 0
KS = 40
KF = 2 * CHUNKS - KS

_mesh = plsc.VectorSubcoreMesh(core_axis_name="c", subcore_axis_name="s")


@functools.partial(
    pl.kernel,
    out_type=jax.ShapeDtypeStruct((NW, NP), jnp.float32),
    mesh=_mesh,
    scratch_types=[
        pltpu.VMEM((CHUNKS, CH), jnp.int32),
        pltpu.VMEM((NP,), jnp.float32),
    ],
    compiler_params=pltpu.CompilerParams(needs_layout_passes=False),
)
def _deg(dst_hbm, out_hbm, idx_v, acc_v):
    c = lax.axis_index("c")
    s = lax.axis_index("s")
    wid = s * NC + c

    def zero_body(i, carry):
        acc_v[pl.ds(i * 16, 16)] = jnp.zeros((16,), jnp.float32)
        return carry

    lax.fori_loop(0, NP // 16, zero_body, 0)
    pltpu.sync_copy(dst_hbm.at[pl.ds(wid * CHUNKS, CHUNKS)], idx_v)

    ones = jnp.ones((16,), jnp.float32)

    def chunk_body(j, carry):
        def vec_body(k, inner):
            dvec = idx_v[j, pl.ds(k * 16, 16)]
            plsc.addupdate_scatter(acc_v, [dvec], ones)
            return inner

        lax.fori_loop(0, CH // 16, vec_body, 0)
        return carry

    lax.fori_loop(0, CHUNKS, chunk_body, 0)
    pltpu.sync_copy(acc_v, out_hbm.at[wid])


@functools.partial(
    pl.kernel,
    out_type=jax.ShapeDtypeStruct((NC, NP, CP), jnp.float32),
    mesh=_mesh,
    scratch_types=[
        pltpu.VMEM((KF * CH,), jnp.int32),
        pltpu.VMEM((KF, CH), jnp.int32),
        [pltpu.VMEM((CH, CP), jnp.float32)] * NBUF,
        pltpu.VMEM((CH, CP), jnp.float32),
        pltpu.VMEM_SHARED((NP, CP), jnp.float32),
        [pltpu.SemaphoreType.DMA] * NBUF,
        [pltpu.SemaphoreType.DMA] * NBUF,
    ],
    compiler_params=pltpu.CompilerParams(use_tc_tiling_on_sc=False),
)
def _hop(src_hbm, dst_hbm, g_hbm, out_hbm, sidx_v, didx_v, rows, zero_v,
         acc_sh, sg, ss):
    c = lax.axis_index("c")
    s = lax.axis_index("s")
    on_slow = c == SLOW_CORE
    chunk_base = jnp.where(on_slow, s * KS, NS * KS + s * KF)
    my_grp = jnp.where(on_slow, KS // NBUF, KF // NBUF)

    pltpu.sync_copy(src_hbm.at[pl.ds(chunk_base * CH, KF * CH)], sidx_v)
    pltpu.sync_copy(dst_hbm.at[pl.ds(chunk_base, KF)], didx_v)

    def gstart(b, j):
        pltpu.async_copy(g_hbm.at[sidx_v.at[pl.ds(j * CH, CH)]], rows[b],
                         sg[b])

    def gwait(b):
        pltpu.make_async_copy(g_hbm.at[sidx_v.at[pl.ds(0, CH)]], rows[b],
                              sg[b]).wait()

    def sstart(b, j):
        pltpu.async_copy(rows[b], acc_sh.at[didx_v.at[j]], ss[b], add=True)

    def swait(b):
        pltpu.make_async_copy(rows[b], acc_sh.at[didx_v.at[0]],
                              ss[b]).wait()

    for b0 in range(LA):
        gstart(b0, b0)

    def zrow(i, carry):
        def zcol(jj, inner):
            zero_v[i, pl.ds(jj * 16, 16)] = jnp.zeros((16,), jnp.float32)
            return inner

        lax.fori_loop(0, CP // 16, zcol, 0)
        return carry

    lax.fori_loop(0, CH, zrow, 0)

    def zacc(i, carry):
        pltpu.sync_copy(zero_v, acc_sh.at[pl.ds(s * RPS + i * CH, CH)])
        return carry

    lax.fori_loop(0, RPS // CH, zacc, 0)
    plsc.subcore_barrier()

    def group_body(j4, carry):
        for b in range(NBUF):
            j = j4 * NBUF + b
            gwait(b)
            sstart(b, j)
            b2 = (b + LA) % NBUF
            if b < LA:
                @pl.when(j4 > 0)
                def _():
                    swait(b2)

                gstart(b2, j + LA)
            else:
                swait(b2)

                @pl.when(j4 < my_grp - 1)
                def _():
                    gstart(b2, j + LA)
        return carry

    lax.fori_loop(0, my_grp, group_body, 0)
    for b0 in range(LA, NBUF):
        swait(b0)
    plsc.subcore_barrier()

    pltpu.sync_copy(acc_sh.at[pl.ds(s * RPS, RPS)],
                    out_hbm.at[c, pl.ds(s * RPS, RPS)])


_BR = 256


def _k1_body(x_ref, wt_ref, dp_ref, g0_ref):
    z = jnp.dot(x_ref[...], wt_ref[...], preferred_element_type=jnp.float32)
    deg = jnp.sum(dp_ref[...], axis=0) + 1.0
    g0_ref[...] = z * lax.rsqrt(deg)[:, None]


def _k1(xp, wt, dp):
    return pl.pallas_call(
        _k1_body,
        grid=(NP // _BR,),
        in_specs=[
            pl.BlockSpec((_BR, D), lambda i: (i, 0)),
            pl.BlockSpec((D, CP), lambda i: (0, 0)),
            pl.BlockSpec((NW, _BR), lambda i: (0, i)),
        ],
        out_specs=pl.BlockSpec((_BR, CP), lambda i: (i, 0)),
        out_shape=jax.ShapeDtypeStruct((NP, CP), jnp.float32),
    )(xp, wt, dp)


def _k2_body(sp_ref, g_ref, dp_ref, o_ref):
    deg = jnp.sum(dp_ref[...], axis=0) + 1.0
    t = sp_ref[0] + sp_ref[1] + g_ref[...]
    o_ref[...] = t / deg[:, None]


def _k2(sp, g, dp):
    return pl.pallas_call(
        _k2_body,
        grid=(NP // _BR,),
        in_specs=[
            pl.BlockSpec((NC, _BR, CP), lambda i: (0, i, 0)),
            pl.BlockSpec((_BR, CP), lambda i: (i, 0)),
            pl.BlockSpec((NW, _BR), lambda i: (0, i)),
        ],
        out_specs=pl.BlockSpec((_BR, CP), lambda i: (i, 0)),
        out_shape=jax.ShapeDtypeStruct((NP, CP), jnp.float32),
    )(sp, g, dp)


def _k3_body(sp_ref, g_ref, dp_ref, b_ref, o_ref):
    deg = jnp.sum(dp_ref[...], axis=0) + 1.0
    dis = lax.rsqrt(deg)
    v = (sp_ref[0] + sp_ref[1] + g_ref[...]) * dis[:, None] + b_ref[...]
    col = lax.broadcasted_iota(jnp.int32, v.shape, 1)
    mask = col < C
    vm = jnp.where(mask, v, -jnp.inf)
    m = jnp.max(vm, axis=1, keepdims=True)
    ex = jnp.where(mask, jnp.exp(v - m), 0.0)
    lse = jnp.log(jnp.sum(ex, axis=1, keepdims=True)) + m
    o_ref[...] = v - lse


def _k3(sp, g, dp, b2):
    return pl.pallas_call(
        _k3_body,
        grid=(NP // _BR,),
        in_specs=[
            pl.BlockSpec((NC, _BR, CP), lambda i: (0, i, 0)),
            pl.BlockSpec((_BR, CP), lambda i: (i, 0)),
            pl.BlockSpec((NW, _BR), lambda i: (0, i)),
            pl.BlockSpec((1, CP), lambda i: (0, 0)),
        ],
        out_specs=pl.BlockSpec((_BR, CP), lambda i: (i, 0)),
        out_shape=jax.ShapeDtypeStruct((NP, CP), jnp.float32),
    )(sp, g, dp, b2)


def kernel(x, edge_index, W, b):
    src = edge_index[0]
    dst = edge_index[1]
    pad = jnp.full((EP - E,), NP - 1, dtype=jnp.int32)
    srcp = jnp.concatenate([src, pad])
    dstp = jnp.concatenate([dst, pad]).reshape(TCH, CH)
    xp = jnp.pad(x, ((0, NP - N), (0, 0)))
    wt = jnp.pad(W.T, ((0, 0), (0, CP - C)))
    b2 = jnp.pad(b, (0, CP - C)).reshape(1, CP)

    dp = _deg(dstp)
    g0 = _k1(xp, wt, dp)
    s1 = _hop(srcp, dstp, g0)
    g1 = _k2(s1, g0, dp)
    s2 = _hop(srcp, dstp, g1)
    out = _k3(s2, g1, dp, b2)
    return out[:N, :C]


# speedup vs baseline: 33.5919x; 1.6520x over previous
import functools

import jax
import jax.numpy as jnp
from jax import lax
from jax.experimental import pallas as pl
from jax.experimental.pallas import tpu as pltpu
from jax.experimental.pallas import tpu_sc as plsc

N = 10000
D = 128
C = 40
E = 320000

NP = 10240
CP = 48
CPH = CP // 2
NC = 2
NS = 16
NW = NC * NS
CH = 128
CHUNKS = 80
EP = NW * CHUNKS * CH
TCH = EP // CH
KPT = TCH // NS
RPS = NP // NS
NBUF = 8
LA = 4
GRP = KPT // NBUF

_mesh = plsc.VectorSubcoreMesh(core_axis_name="c", subcore_axis_name="s")


@functools.partial(
    pl.kernel,
    out_type=jax.ShapeDtypeStruct((NW, NP), jnp.float32),
    mesh=_mesh,
    scratch_types=[
        pltpu.VMEM((CHUNKS, CH), jnp.int32),
        pltpu.VMEM((NP,), jnp.float32),
    ],
    compiler_params=pltpu.CompilerParams(needs_layout_passes=False),
)
def _deg(dst_hbm, out_hbm, idx_v, acc_v):
    c = lax.axis_index("c")
    s = lax.axis_index("s")
    wid = s * NC + c

    def zero_body(i, carry):
        acc_v[pl.ds(i * 16, 16)] = jnp.zeros((16,), jnp.float32)
        return carry

    lax.fori_loop(0, NP // 16, zero_body, 0)
    pltpu.sync_copy(dst_hbm.at[pl.ds(wid * CHUNKS, CHUNKS)], idx_v)

    ones = jnp.ones((16,), jnp.float32)

    def chunk_body(j, carry):
        def vec_body(k, inner):
            dvec = idx_v[j, pl.ds(k * 16, 16)]
            plsc.addupdate_scatter(acc_v, [dvec], ones)
            return inner

        lax.fori_loop(0, CH // 16, vec_body, 0)
        return carry

    lax.fori_loop(0, CHUNKS, chunk_body, 0)
    pltpu.sync_copy(acc_v, out_hbm.at[wid])


@functools.partial(
    pl.kernel,
    out_type=jax.ShapeDtypeStruct((NC, NP, CPH), jnp.float32),
    mesh=_mesh,
    scratch_types=[
        pltpu.VMEM((KPT * CH,), jnp.int32),
        pltpu.VMEM((KPT, CH), jnp.int32),
        [pltpu.VMEM((CH, CPH), jnp.float32)] * NBUF,
        pltpu.VMEM((CH, CPH), jnp.float32),
        pltpu.VMEM_SHARED((NP, CPH), jnp.float32),
        pltpu.VMEM_SHARED((NP, CPH), jnp.float32),
        [pltpu.SemaphoreType.DMA] * NBUF,
        [pltpu.SemaphoreType.DMA] * NBUF,
    ],
    compiler_params=pltpu.CompilerParams(use_tc_tiling_on_sc=False),
)
def _hop(src_hbm, dst_hbm, g_hbm, out_hbm, sidx_v, didx_v, rows, zero_v,
         acc_sh, g_sh, sg, ss):
    c = lax.axis_index("c")
    s = lax.axis_index("s")

    pltpu.sync_copy(src_hbm.at[pl.ds(s * (KPT * CH), KPT * CH)], sidx_v)
    pltpu.sync_copy(dst_hbm.at[pl.ds(s * KPT, KPT)], didx_v)
    pltpu.sync_copy(g_hbm.at[c, pl.ds(s * RPS, RPS)],
                    g_sh.at[pl.ds(s * RPS, RPS)])

    def gstart(b, j):
        pltpu.async_copy(g_sh.at[sidx_v.at[pl.ds(j * CH, CH)]], rows[b],
                         sg[b])

    def gwait(b):
        pltpu.make_async_copy(g_sh.at[sidx_v.at[pl.ds(0, CH)]], rows[b],
                              sg[b]).wait()

    def sstart(b, j):
        pltpu.async_copy(rows[b], acc_sh.at[didx_v.at[j]], ss[b], add=True)

    def swait(b):
        pltpu.make_async_copy(rows[b], acc_sh.at[didx_v.at[0]],
                              ss[b]).wait()

    def zrow(i, carry):
        zero_v[i, pl.ds(0, 16)] = jnp.zeros((16,), jnp.float32)
        zero_v[i, pl.ds(CPH - 16, 16)] = jnp.zeros((16,), jnp.float32)
        return carry

    lax.fori_loop(0, CH, zrow, 0)

    def zacc(i, carry):
        pltpu.sync_copy(zero_v, acc_sh.at[pl.ds(s * RPS + i * CH, CH)])
        return carry

    lax.fori_loop(0, RPS // CH, zacc, 0)
    plsc.subcore_barrier()

    for b0 in range(LA):
        gstart(b0, b0)

    def group_body(j4, carry):
        for b in range(NBUF):
            j = j4 * NBUF + b
            gwait(b)
            sstart(b, j)
            b2 = (b + LA) % NBUF
            if b < LA:
                @pl.when(j4 > 0)
                def _():
                    swait(b2)

                gstart(b2, j + LA)
            else:
                swait(b2)

                @pl.when(j4 < GRP - 1)
                def _():
                    gstart(b2, j + LA)
        return carry

    lax.fori_loop(0, GRP, group_body, 0)
    for b0 in range(LA, NBUF):
        swait(b0)
    plsc.subcore_barrier()

    pltpu.sync_copy(acc_sh.at[pl.ds(s * RPS, RPS)],
                    out_hbm.at[c, pl.ds(s * RPS, RPS)])


_BR = 256


def _k1_body(x_ref, wt_ref, dp_ref, g0_ref):
    z = jnp.dot(x_ref[...], wt_ref[...], preferred_element_type=jnp.float32)
    deg = jnp.sum(dp_ref[...], axis=0) + 1.0
    g0 = z * lax.rsqrt(deg)[:, None]
    g0_ref[0] = g0[:, :CPH]
    g0_ref[1] = g0[:, CPH:]


def _k1(xp, wt, dp):
    return pl.pallas_call(
        _k1_body,
        grid=(NP // _BR,),
        in_specs=[
            pl.BlockSpec((_BR, D), lambda i: (i, 0)),
            pl.BlockSpec((D, CP), lambda i: (0, 0)),
            pl.BlockSpec((NW, _BR), lambda i: (0, i)),
        ],
        out_specs=pl.BlockSpec((NC, _BR, CPH), lambda i: (0, i, 0)),
        out_shape=jax.ShapeDtypeStruct((NC, NP, CPH), jnp.float32),
    )(xp, wt, dp)


def _k2_body(sp_ref, g_ref, dp_ref, o_ref):
    deg = jnp.sum(dp_ref[...], axis=0) + 1.0
    minv = (1.0 / deg)[:, None]
    o_ref[0] = (sp_ref[0] + g_ref[0]) * minv
    o_ref[1] = (sp_ref[1] + g_ref[1]) * minv


def _k2(sp, g, dp):
    return pl.pallas_call(
        _k2_body,
        grid=(NP // _BR,),
        in_specs=[
            pl.BlockSpec((NC, _BR, CPH), lambda i: (0, i, 0)),
            pl.BlockSpec((NC, _BR, CPH), lambda i: (0, i, 0)),
            pl.BlockSpec((NW, _BR), lambda i: (0, i)),
        ],
        out_specs=pl.BlockSpec((NC, _BR, CPH), lambda i: (0, i, 0)),
        out_shape=jax.ShapeDtypeStruct((NC, NP, CPH), jnp.float32),
    )(sp, g, dp)


def _k3_body(sp_ref, g_ref, dp_ref, b_ref, o_ref):
    deg = jnp.sum(dp_ref[...], axis=0) + 1.0
    dis = lax.rsqrt(deg)
    t = jnp.concatenate([sp_ref[0] + g_ref[0], sp_ref[1] + g_ref[1]], axis=1)
    v = t * dis[:, None] + b_ref[...]
    col = lax.broadcasted_iota(jnp.int32, v.shape, 1)
    mask = col < C
    vm = jnp.where(mask, v, -jnp.inf)
    m = jnp.max(vm, axis=1, keepdims=True)
    ex = jnp.where(mask, jnp.exp(v - m), 0.0)
    lse = jnp.log(jnp.sum(ex, axis=1, keepdims=True)) + m
    o_ref[...] = v - lse


def _k3(sp, g, dp, b2):
    return pl.pallas_call(
        _k3_body,
        grid=(NP // _BR,),
        in_specs=[
            pl.BlockSpec((NC, _BR, CPH), lambda i: (0, i, 0)),
            pl.BlockSpec((NC, _BR, CPH), lambda i: (0, i, 0)),
            pl.BlockSpec((NW, _BR), lambda i: (0, i)),
            pl.BlockSpec((1, CP), lambda i: (0, 0)),
        ],
        out_specs=pl.BlockSpec((_BR, CP), lambda i: (i, 0)),
        out_shape=jax.ShapeDtypeStruct((NP, CP), jnp.float32),
    )(sp, g, dp, b2)


def kernel(x, edge_index, W, b):
    src = edge_index[0]
    dst = edge_index[1]
    pad = jnp.full((EP - E,), NP - 1, dtype=jnp.int32)
    srcp = jnp.concatenate([src, pad])
    dstp = jnp.concatenate([dst, pad]).reshape(TCH, CH)
    xp = jnp.pad(x, ((0, NP - N), (0, 0)))
    wt = jnp.pad(W.T, ((0, 0), (0, CP - C)))
    b2 = jnp.pad(b, (0, CP - C)).reshape(1, CP)

    dp = _deg(dstp)
    g0 = _k1(xp, wt, dp)
    s1 = _hop(srcp, dstp, g0)
    g1 = _k2(s1, g0, dp)
    s2 = _hop(srcp, dstp, g1)
    out = _k3(s2, g1, dp, b2)
    return out[:N, :C]
